# baseline (speedup 1.0000x reference)
import functools

import jax
import jax.numpy as jnp
from jax import lax
from jax.experimental import pallas as pl
from jax.experimental.pallas import tpu as pltpu
from jax.experimental.pallas import tpu_sc as plsc

D_MODEL = 1024
T = 8192

_NC = 2
_NS = 16
_NW = _NC * _NS
_BPW = T // _NW
_C = 16
_NCHUNK = _BPW // _C
_NBUF = 6


@functools.partial(
    pl.kernel,
    mesh=plsc.VectorSubcoreMesh(core_axis_name="c", subcore_axis_name="s"),
    out_type=jax.ShapeDtypeStruct((1, T, D_MODEL), jnp.float32),
    scratch_types=(
        [pltpu.VMEM((_BPW,), jnp.int32)]
        + [pltpu.VMEM((_C, D_MODEL), jnp.float32)] * _NBUF
        + [pltpu.SemaphoreType.DMA] * (2 * _NBUF)
    ),
)
def _gather_rows(types_hbm, emb_hbm, out_hbm, idx_v, *rest):
    bufs = rest[:_NBUF]
    gsem = rest[_NBUF:2 * _NBUF]
    wsem = rest[2 * _NBUF:]
    wid = lax.axis_index("s") * _NC + lax.axis_index("c")
    base = wid * _BPW

    def gather(c):
        return pltpu.async_copy(
            emb_hbm.at[idx_v.at[pl.ds(c * _C, _C)]],
            bufs[c % _NBUF], gsem[c % _NBUF])

    pltpu.sync_copy(types_hbm.at[pl.ds(base, _C)], idx_v.at[pl.ds(0, _C)])
    gh = [None] * _NCHUNK
    wh = [None] * _NCHUNK
    gh[0] = gather(0)
    pltpu.sync_copy(types_hbm.at[pl.ds(base + _C, _BPW - _C)],
                    idx_v.at[pl.ds(_C, _BPW - _C)])
    for c in range(1, _NBUF - 1):
        gh[c] = gather(c)
    for c in range(_NCHUNK):
        n = c + _NBUF - 1
        if n < _NCHUNK:
            if c >= 1:
                wh[c - 1].wait()
            gh[n] = gather(n)
        gh[c].wait()
        wh[c] = pltpu.async_copy(
            bufs[c % _NBUF], out_hbm.at[0, pl.ds(base + c * _C, _C)],
            wsem[c % _NBUF])
    for c in range(_NCHUNK - _NBUF, _NCHUNK):
        wh[c].wait()


def kernel(types, emb):
    return _gather_rows(types.astype(jnp.int32), emb)


# speedup vs baseline: 1.0097x; 1.0097x over previous
import functools

import jax
import jax.numpy as jnp
from jax import lax
from jax.experimental import pallas as pl
from jax.experimental.pallas import tpu as pltpu
from jax.experimental.pallas import tpu_sc as plsc

D_MODEL = 1024
T = 8192

_NC = 2
_NS = 16
_NW = _NC * _NS
_BPW = T // _NW
_C = 32
_NCHUNK = _BPW // _C
_NBUF = 3


@functools.partial(
    pl.kernel,
    mesh=plsc.VectorSubcoreMesh(core_axis_name="c", subcore_axis_name="s"),
    out_type=jax.ShapeDtypeStruct((1, T, D_MODEL), jnp.float32),
    scratch_types=(
        [pltpu.VMEM((_BPW,), jnp.int32)]
        + [pltpu.VMEM((_C, D_MODEL), jnp.float32)] * _NBUF
        + [pltpu.SemaphoreType.DMA] * (2 * _NBUF)
    ),
)
def _gather_rows(types_hbm, emb_hbm, out_hbm, idx_v, *rest):
    bufs = rest[:_NBUF]
    gsem = rest[_NBUF:2 * _NBUF]
    wsem = rest[2 * _NBUF:]
    wid = lax.axis_index("s") * _NC + lax.axis_index("c")
    base = wid * _BPW

    def gather(c):
        return pltpu.async_copy(
            emb_hbm.at[idx_v.at[pl.ds(c * _C, _C)]],
            bufs[c % _NBUF], gsem[c % _NBUF])

    pltpu.sync_copy(types_hbm.at[pl.ds(base, _C)], idx_v.at[pl.ds(0, _C)])
    gh = [None] * _NCHUNK
    wh = [None] * _NCHUNK
    gh[0] = gather(0)
    pltpu.sync_copy(types_hbm.at[pl.ds(base + _C, _C)],
                    idx_v.at[pl.ds(_C, _C)])
    gh[1] = gather(1)
    pltpu.sync_copy(types_hbm.at[pl.ds(base + 2 * _C, _BPW - 2 * _C)],
                    idx_v.at[pl.ds(2 * _C, _BPW - 2 * _C)])
    for c in range(2, _NBUF - 1):
        gh[c] = gather(c)
    for c in range(_NCHUNK):
        n = c + _NBUF - 1
        if n < _NCHUNK:
            if c >= 1:
                wh[c - 1].wait()
            gh[n] = gather(n)
        gh[c].wait()
        wh[c] = pltpu.async_copy(
            bufs[c % _NBUF], out_hbm.at[0, pl.ds(base + c * _C, _C)],
            wsem[c % _NBUF])
    for c in range(_NCHUNK - _NBUF, _NCHUNK):
        wh[c].wait()


def kernel(types, emb):
    return _gather_rows(types.astype(jnp.int32), emb)
